# Initial kernel scaffold; baseline (speedup 1.0000x reference)
import jax
import jax.numpy as jnp
from jax.experimental import pallas as pl


def kernel(src_x, tgt_x, edge_index, W_src, b_src, W_tgt, b_tgt, Wl0, bl0, Wr0, Wl1, bl1, Wr1):
    raise NotImplementedError("write your pallas kernel here")


# speedup vs baseline: 9.3069x; 9.3069x over previous
import functools

import jax
import jax.numpy as jnp
from jax import lax
from jax.experimental import pallas as pl
from jax.experimental.pallas import tpu as pltpu
from jax.experimental.pallas import tpu_sc as plsc

N_SRC_ROWS = 5000
NNODE = 10000
NEDGE = 320000
HD = 128
NC = 2
NS = 16
NW = NC * NS
EPW = NEDGE // NW
CH = 80
NCHK = EPW // CH
NSB = 5
SB = NCHK // NSB
RPTA = 624
TAIL = NNODE - NS * RPTA
CW = 16
ROWB = 1000
GRID = NNODE // ROWB


def _make_sc_agg(with_count: bool):
    mesh = plsc.VectorSubcoreMesh(
        core_axis_name="c", subcore_axis_name="s",
        num_cores=NC, num_subcores=NS)
    out_type = [jax.ShapeDtypeStruct((NC, NNODE, HD), jnp.float32)]
    scratch = [
        pltpu.VMEM((SB, CH), jnp.int32),
        pltpu.VMEM((SB, CH), jnp.int32),
        pltpu.VMEM((CH, HD), jnp.float32),
        pltpu.VMEM((CH, HD), jnp.float32),
        pltpu.VMEM_SHARED((NNODE, HD), jnp.float32),
        pltpu.SemaphoreType.DMA,
        pltpu.SemaphoreType.DMA,
    ]
    if with_count:
        out_type.append(jax.ShapeDtypeStruct((NNODE,), jnp.float32))
        out_type.append(jax.ShapeDtypeStruct((NNODE,), jnp.float32))
        scratch += [
            pltpu.VMEM((CH,), jnp.float32),
            pltpu.VMEM_SHARED((NNODE,), jnp.float32),
            pltpu.VMEM((RPTA,), jnp.float32),
        ]

    def body(*refs):
        if with_count:
            (x_hbm, srcw, dstw, zf_hbm, zc_hbm, ones_hbm, out_hbm,
             cnt0_hbm, cnt1_hbm,
             sidx, didx, bufa, bufb, acc, sema, semb, onesb, accc,
             cbuf) = refs
        else:
            (x_hbm, srcw, dstw, zf_hbm, out_hbm,
             sidx, didx, bufa, bufb, acc, sema, semb) = refs
        cid = lax.axis_index("c")
        sid = lax.axis_index("s")
        wid = sid * NC + cid

        base = sid * RPTA
        tail = NS * RPTA
        pltpu.sync_copy(zf_hbm.at[pl.ds(base, RPTA)],
                        acc.at[pl.ds(base, RPTA)])
        if with_count:
            pltpu.sync_copy(zc_hbm.at[pl.ds(base, RPTA)], cbuf)
            pltpu.sync_copy(cbuf, accc.at[pl.ds(base, RPTA)])
            pltpu.sync_copy(ones_hbm, onesb)

        @pl.when(sid == NS - 1)
        def _zero_tail():
            pltpu.sync_copy(zf_hbm.at[pl.ds(tail, TAIL)],
                            acc.at[pl.ds(tail, TAIL)])
            if with_count:
                pltpu.sync_copy(cbuf.at[pl.ds(0, TAIL)],
                                accc.at[pl.ds(tail, TAIL)])

        plsc.subcore_barrier()

        def wait_a():
            pltpu.make_async_copy(x_hbm.at[sidx.at[0]], bufa, sema).wait()

        def wait_b():
            pltpu.make_async_copy(x_hbm.at[sidx.at[0]], bufb, semb).wait()

        def scat(buf, j):
            pltpu.sync_copy(buf, acc.at[didx.at[j]], add=True)
            if with_count:
                pltpu.sync_copy(onesb, accc.at[didx.at[j]], add=True)

        def super_body(sb, carry):
            pltpu.sync_copy(srcw.at[wid, sb], sidx)
            pltpu.sync_copy(dstw.at[wid, sb], didx)
            pltpu.async_copy(x_hbm.at[sidx.at[0]], bufa, sema)

            def pair(jj, c):
                a = jj * 2
                pltpu.async_copy(x_hbm.at[sidx.at[a + 1]], bufb, semb)
                wait_a()
                scat(bufa, a)
                pltpu.async_copy(x_hbm.at[sidx.at[a + 2]], bufa, sema)
                wait_b()
                scat(bufb, a + 1)
                return c

            lax.fori_loop(0, (SB - 1) // 2, pair, 0)
            wait_a()
            scat(bufa, SB - 1)
            return carry

        lax.fori_loop(0, NSB, super_body, 0)

        plsc.subcore_barrier()
        pltpu.sync_copy(acc.at[pl.ds(base, RPTA)],
                        out_hbm.at[cid, pl.ds(base, RPTA)])
        if with_count:
            pltpu.sync_copy(accc.at[pl.ds(base, RPTA)], cbuf)

            @pl.when(cid == 0)
            def _write_cnt0():
                pltpu.sync_copy(cbuf, cnt0_hbm.at[pl.ds(base, RPTA)])

            @pl.when(cid == 1)
            def _write_cnt1():
                pltpu.sync_copy(cbuf, cnt1_hbm.at[pl.ds(base, RPTA)])

        @pl.when(sid == NS - 1)
        def _write_tail():
            pltpu.sync_copy(acc.at[pl.ds(tail, TAIL)],
                            out_hbm.at[cid, pl.ds(tail, TAIL)])
            if with_count:
                pltpu.sync_copy(accc.at[pl.ds(tail, TAIL)],
                                cbuf.at[pl.ds(0, TAIL)])

                @pl.when(cid == 0)
                def _write_cnt0_tail():
                    pltpu.sync_copy(cbuf.at[pl.ds(0, TAIL)],
                                    cnt0_hbm.at[pl.ds(tail, TAIL)])

                @pl.when(cid == 1)
                def _write_cnt1_tail():
                    pltpu.sync_copy(cbuf.at[pl.ds(0, TAIL)],
                                    cnt1_hbm.at[pl.ds(tail, TAIL)])

    return pl.kernel(body, out_type=out_type, mesh=mesh,
                     scratch_types=scratch)


_sc_agg_count = _make_sc_agg(True)
_sc_agg_plain = _make_sc_agg(False)


def _proj_body(x_ref, ws_ref, bs_ref, wt_ref, bt_ref, o_ref):
    use_src = pl.program_id(0) < (N_SRC_ROWS // ROWB)
    w = jnp.where(use_src, ws_ref[...], wt_ref[...])
    b = jnp.where(use_src, bs_ref[...], bt_ref[...])
    o_ref[...] = jnp.dot(x_ref[...], w, preferred_element_type=jnp.float32) + b


_proj = pl.pallas_call(
    _proj_body,
    grid=(GRID,),
    in_specs=[
        pl.BlockSpec((ROWB, HD), lambda i: (i, 0)),
        pl.BlockSpec((HD, HD), lambda i: (0, 0)),
        pl.BlockSpec((1, HD), lambda i: (0, 0)),
        pl.BlockSpec((HD, HD), lambda i: (0, 0)),
        pl.BlockSpec((1, HD), lambda i: (0, 0)),
    ],
    out_specs=pl.BlockSpec((ROWB, HD), lambda i: (i, 0)),
    out_shape=jax.ShapeDtypeStruct((NNODE, HD), jnp.float32),
)


def _layer0_body(p0, p1, c0, c1, h_ref, wl, bl, wr, o_ref, cnt_o):
    cnt = jnp.maximum(c0[...] + c1[...], 1.0)
    agg = (p0[...] + p1[...]) / cnt
    out = (jnp.dot(agg, wl[...], preferred_element_type=jnp.float32)
           + bl[...]
           + jnp.dot(h_ref[...], wr[...], preferred_element_type=jnp.float32))
    o_ref[...] = jnp.maximum(out, 0.0)
    cnt_o[...] = cnt


_layer0 = pl.pallas_call(
    _layer0_body,
    grid=(GRID,),
    in_specs=[
        pl.BlockSpec((ROWB, HD), lambda i: (i, 0)),
        pl.BlockSpec((ROWB, HD), lambda i: (i, 0)),
        pl.BlockSpec((ROWB, 1), lambda i: (i, 0)),
        pl.BlockSpec((ROWB, 1), lambda i: (i, 0)),
        pl.BlockSpec((ROWB, HD), lambda i: (i, 0)),
        pl.BlockSpec((HD, HD), lambda i: (0, 0)),
        pl.BlockSpec((1, HD), lambda i: (0, 0)),
        pl.BlockSpec((HD, HD), lambda i: (0, 0)),
    ],
    out_specs=[
        pl.BlockSpec((ROWB, HD), lambda i: (i, 0)),
        pl.BlockSpec((ROWB, 1), lambda i: (i, 0)),
    ],
    out_shape=[
        jax.ShapeDtypeStruct((NNODE, HD), jnp.float32),
        jax.ShapeDtypeStruct((NNODE, 1), jnp.float32),
    ],
)


def _layer1_body(p0, p1, cnt_ref, h_ref, wl, bl, wr, o_ref):
    agg = (p0[...] + p1[...]) / cnt_ref[...]
    o_ref[...] = (jnp.dot(agg, wl[...], preferred_element_type=jnp.float32)
                  + bl[...]
                  + jnp.dot(h_ref[...], wr[...],
                            preferred_element_type=jnp.float32))


_layer1 = pl.pallas_call(
    _layer1_body,
    grid=(GRID,),
    in_specs=[
        pl.BlockSpec((ROWB, HD), lambda i: (i, 0)),
        pl.BlockSpec((ROWB, HD), lambda i: (i, 0)),
        pl.BlockSpec((ROWB, 1), lambda i: (i, 0)),
        pl.BlockSpec((ROWB, HD), lambda i: (i, 0)),
        pl.BlockSpec((HD, HD), lambda i: (0, 0)),
        pl.BlockSpec((1, HD), lambda i: (0, 0)),
        pl.BlockSpec((HD, HD), lambda i: (0, 0)),
    ],
    out_specs=pl.BlockSpec((ROWB, HD), lambda i: (i, 0)),
    out_shape=jax.ShapeDtypeStruct((NNODE, HD), jnp.float32),
)


def kernel(src_x, tgt_x, edge_index, W_src, b_src, W_tgt, b_tgt,
           Wl0, bl0, Wr0, Wl1, bl1, Wr1):
    xin = jnp.concatenate([src_x, tgt_x], axis=0)
    srcw = edge_index[0].reshape(NW, NSB, SB, CH)
    dstw = edge_index[1].reshape(NW, NSB, SB, CH)
    zf = jnp.zeros((NNODE, HD), jnp.float32)
    zc = jnp.zeros((NNODE,), jnp.float32)
    ones = jnp.ones((CH,), jnp.float32)

    x = _proj(xin, W_src.T, b_src[None], W_tgt.T, b_tgt[None])
    p, c0, c1 = _sc_agg_count(x, srcw, dstw, zf, zc, ones)
    h1, cnt = _layer0(p[0], p[1], c0.reshape(NNODE, 1), c1.reshape(NNODE, 1),
                      x, Wl0.T, bl0[None], Wr0.T)
    (p2,) = _sc_agg_plain(h1, srcw, dstw, zf)
    h2 = _layer1(p2[0], p2[1], cnt, h1, Wl1.T, bl1[None], Wr1.T)
    return h2


# speedup vs baseline: 9.7489x; 1.0475x over previous
import functools

import jax
import jax.numpy as jnp
from jax import lax
from jax.experimental import pallas as pl
from jax.experimental.pallas import tpu as pltpu
from jax.experimental.pallas import tpu_sc as plsc

N_SRC_ROWS = 5000
NNODE = 10000
NEDGE = 320000
HD = 128
NC = 2
NS = 16
NW = NC * NS
EPW = NEDGE // NW
CH = 80
NCHK = EPW // CH
NSB = 5
SB = NCHK // NSB
RPTA = 624
TAIL = NNODE - NS * RPTA
CW = 16
ROWB = 1000
GRID = NNODE // ROWB


def _make_sc_agg(with_count: bool):
    mesh = plsc.VectorSubcoreMesh(
        core_axis_name="c", subcore_axis_name="s",
        num_cores=NC, num_subcores=NS)
    out_type = [jax.ShapeDtypeStruct((NC, NNODE, HD), jnp.float32)]
    scratch = [
        pltpu.VMEM((SB, CH), jnp.int32),
        pltpu.VMEM((SB, CH), jnp.int32),
        pltpu.VMEM((CH, HD), jnp.float32),
        pltpu.VMEM((CH, HD), jnp.float32),
        pltpu.VMEM_SHARED((NNODE, HD), jnp.float32),
        pltpu.SemaphoreType.DMA,
        pltpu.SemaphoreType.DMA,
    ]
    if with_count:
        out_type.append(jax.ShapeDtypeStruct((NNODE,), jnp.float32))
        out_type.append(jax.ShapeDtypeStruct((NNODE,), jnp.float32))
        scratch += [
            pltpu.VMEM((CH,), jnp.float32),
            pltpu.VMEM_SHARED((NNODE,), jnp.float32),
            pltpu.VMEM((RPTA,), jnp.float32),
        ]

    def body(*refs):
        if with_count:
            (x_hbm, srcw, dstw, zf_hbm, zc_hbm, ones_hbm, out_hbm,
             cnt0_hbm, cnt1_hbm,
             sidx, didx, bufa, bufb, acc, sema, semb, onesb, accc,
             cbuf) = refs
        else:
            (x_hbm, srcw, dstw, zf_hbm, out_hbm,
             sidx, didx, bufa, bufb, acc, sema, semb) = refs
        cid = lax.axis_index("c")
        sid = lax.axis_index("s")
        wid = sid * NC + cid

        base = sid * RPTA
        tail = NS * RPTA
        pltpu.sync_copy(zf_hbm.at[pl.ds(base, RPTA)],
                        acc.at[pl.ds(base, RPTA)])
        if with_count:
            pltpu.sync_copy(zc_hbm.at[pl.ds(base, RPTA)], cbuf)
            pltpu.sync_copy(cbuf, accc.at[pl.ds(base, RPTA)])
            pltpu.sync_copy(ones_hbm, onesb)

        @pl.when(sid == NS - 1)
        def _zero_tail():
            pltpu.sync_copy(zf_hbm.at[pl.ds(tail, TAIL)],
                            acc.at[pl.ds(tail, TAIL)])
            if with_count:
                pltpu.sync_copy(cbuf.at[pl.ds(0, TAIL)],
                                accc.at[pl.ds(tail, TAIL)])

        plsc.subcore_barrier()

        def wait_a():
            pltpu.make_async_copy(x_hbm.at[sidx.at[0]], bufa, sema).wait()

        def wait_b():
            pltpu.make_async_copy(x_hbm.at[sidx.at[0]], bufb, semb).wait()

        def scat(buf, j):
            pltpu.sync_copy(buf, acc.at[didx.at[j]], add=True)
            if with_count:
                pltpu.sync_copy(onesb, accc.at[didx.at[j]], add=True)

        def super_body(sb, carry):
            pltpu.sync_copy(srcw.at[wid, sb], sidx)
            pltpu.sync_copy(dstw.at[wid, sb], didx)
            pltpu.async_copy(x_hbm.at[sidx.at[0]], bufa, sema)

            def pair(jj, c):
                a = jj * 2
                pltpu.async_copy(x_hbm.at[sidx.at[a + 1]], bufb, semb)
                wait_a()
                scat(bufa, a)
                pltpu.async_copy(x_hbm.at[sidx.at[a + 2]], bufa, sema)
                wait_b()
                scat(bufb, a + 1)
                return c

            lax.fori_loop(0, (SB - 1) // 2, pair, 0)
            wait_a()
            scat(bufa, SB - 1)
            return carry

        lax.fori_loop(0, NSB, super_body, 0)

        plsc.subcore_barrier()
        pltpu.sync_copy(acc.at[pl.ds(base, RPTA)],
                        out_hbm.at[cid, pl.ds(base, RPTA)])
        if with_count:
            pltpu.sync_copy(accc.at[pl.ds(base, RPTA)], cbuf)

            @pl.when(cid == 0)
            def _write_cnt0():
                pltpu.sync_copy(cbuf, cnt0_hbm.at[pl.ds(base, RPTA)])

            @pl.when(cid == 1)
            def _write_cnt1():
                pltpu.sync_copy(cbuf, cnt1_hbm.at[pl.ds(base, RPTA)])

        @pl.when(sid == NS - 1)
        def _write_tail():
            pltpu.sync_copy(acc.at[pl.ds(tail, TAIL)],
                            out_hbm.at[cid, pl.ds(tail, TAIL)])
            if with_count:
                pltpu.sync_copy(accc.at[pl.ds(tail, TAIL)],
                                cbuf.at[pl.ds(0, TAIL)])

                @pl.when(cid == 0)
                def _write_cnt0_tail():
                    pltpu.sync_copy(cbuf.at[pl.ds(0, TAIL)],
                                    cnt0_hbm.at[pl.ds(tail, TAIL)])

                @pl.when(cid == 1)
                def _write_cnt1_tail():
                    pltpu.sync_copy(cbuf.at[pl.ds(0, TAIL)],
                                    cnt1_hbm.at[pl.ds(tail, TAIL)])

    return pl.kernel(body, out_type=out_type, mesh=mesh,
                     scratch_types=scratch)


_sc_agg_count = _make_sc_agg(True)
_sc_agg_plain = _make_sc_agg(False)


_HALF_BLKS = N_SRC_ROWS // ROWB


def _proj_body(xs_ref, xt_ref, ws_ref, bs_ref, wt_ref, bt_ref, o_ref):
    use_src = pl.program_id(0) < _HALF_BLKS
    x = jnp.where(use_src, xs_ref[...], xt_ref[...])
    w = jnp.where(use_src, ws_ref[...], wt_ref[...])
    b = jnp.where(use_src, bs_ref[...], bt_ref[...])
    o_ref[...] = jnp.dot(x, w, preferred_element_type=jnp.float32) + b


_proj = pl.pallas_call(
    _proj_body,
    grid=(GRID,),
    in_specs=[
        pl.BlockSpec((ROWB, HD), lambda i: (jnp.minimum(i, _HALF_BLKS - 1), 0)),
        pl.BlockSpec((ROWB, HD),
                     lambda i: (jnp.maximum(i - _HALF_BLKS, 0), 0)),
        pl.BlockSpec((HD, HD), lambda i: (0, 0)),
        pl.BlockSpec((1, HD), lambda i: (0, 0)),
        pl.BlockSpec((HD, HD), lambda i: (0, 0)),
        pl.BlockSpec((1, HD), lambda i: (0, 0)),
    ],
    out_specs=pl.BlockSpec((ROWB, HD), lambda i: (i, 0)),
    out_shape=jax.ShapeDtypeStruct((NNODE, HD), jnp.float32),
)


def _layer_body(relu, p_ref, c0, c1, h_ref, wl, bl, wr, o_ref):
    cnt = jnp.maximum(c0[...] + c1[...], 1.0)
    agg = (p_ref[0] + p_ref[1]) / cnt
    out = (jnp.dot(agg, wl[...], preferred_element_type=jnp.float32)
           + bl[...]
           + jnp.dot(h_ref[...], wr[...], preferred_element_type=jnp.float32))
    o_ref[...] = jnp.maximum(out, 0.0) if relu else out


def _make_layer(relu):
    return pl.pallas_call(
        functools.partial(_layer_body, relu),
        grid=(GRID,),
        in_specs=[
            pl.BlockSpec((2, ROWB, HD), lambda i: (0, i, 0)),
            pl.BlockSpec((ROWB, 1), lambda i: (i, 0)),
            pl.BlockSpec((ROWB, 1), lambda i: (i, 0)),
            pl.BlockSpec((ROWB, HD), lambda i: (i, 0)),
            pl.BlockSpec((HD, HD), lambda i: (0, 0)),
            pl.BlockSpec((1, HD), lambda i: (0, 0)),
            pl.BlockSpec((HD, HD), lambda i: (0, 0)),
        ],
        out_specs=pl.BlockSpec((ROWB, HD), lambda i: (i, 0)),
        out_shape=jax.ShapeDtypeStruct((NNODE, HD), jnp.float32),
    )


_layer0 = _make_layer(True)
_layer1 = _make_layer(False)


def kernel(src_x, tgt_x, edge_index, W_src, b_src, W_tgt, b_tgt,
           Wl0, bl0, Wr0, Wl1, bl1, Wr1):
    srcw = edge_index[0].reshape(NW, NSB, SB, CH)
    dstw = edge_index[1].reshape(NW, NSB, SB, CH)
    zf = jnp.zeros((NNODE, HD), jnp.float32)
    zc = jnp.zeros((NNODE,), jnp.float32)
    ones = jnp.ones((CH,), jnp.float32)

    x = _proj(src_x, tgt_x, W_src.T, b_src[None], W_tgt.T, b_tgt[None])
    p, c0, c1 = _sc_agg_count(x, srcw, dstw, zf, zc, ones)
    c0r = c0.reshape(NNODE, 1)
    c1r = c1.reshape(NNODE, 1)
    h1 = _layer0(p, c0r, c1r, x, Wl0.T, bl0[None], Wr0.T)
    (p2,) = _sc_agg_plain(h1, srcw, dstw, zf)
    h2 = _layer1(p2, c0r, c1r, h1, Wl1.T, bl1[None], Wr1.T)
    return h2


# speedup vs baseline: 10.3383x; 1.0605x over previous
import functools

import jax
import jax.numpy as jnp
from jax import lax
from jax.experimental import pallas as pl
from jax.experimental.pallas import tpu as pltpu
from jax.experimental.pallas import tpu_sc as plsc

N_SRC_ROWS = 5000
NNODE = 10000
NEDGE = 320000
HD = 128
NC = 2
NS = 16
NW = NC * NS
EPW = NEDGE // NW
CH = 128
PADW = 240
EPWP = EPW + PADW
NCHK = EPWP // CH
NSB = 5
SB = NCHK // NSB
NDUM = 1024
NND = NNODE + NDUM
RPTA = 624
TAIL = NNODE - NS * RPTA
CW = 16
ROWB = 1000
GRID = NNODE // ROWB


def _make_sc_agg(with_count: bool):
    mesh = plsc.VectorSubcoreMesh(
        core_axis_name="c", subcore_axis_name="s",
        num_cores=NC, num_subcores=NS)
    out_type = [jax.ShapeDtypeStruct((NC, NNODE, HD), jnp.float32)]
    scratch = [
        pltpu.VMEM((SB, CH), jnp.int32),
        pltpu.VMEM((SB, CH), jnp.int32),
        pltpu.VMEM((CH, HD), jnp.float32),
        pltpu.VMEM((CH, HD), jnp.float32),
        pltpu.VMEM_SHARED((NND, HD), jnp.float32),
        pltpu.SemaphoreType.DMA,
        pltpu.SemaphoreType.DMA,
    ]
    if with_count:
        out_type.append(jax.ShapeDtypeStruct((NNODE,), jnp.float32))
        out_type.append(jax.ShapeDtypeStruct((NNODE,), jnp.float32))
        scratch += [
            pltpu.VMEM((CH,), jnp.float32),
            pltpu.VMEM_SHARED((NND,), jnp.float32),
            pltpu.VMEM((RPTA,), jnp.float32),
        ]

    def body(*refs):
        if with_count:
            (x_hbm, srcw, dstw, zf_hbm, zc_hbm, ones_hbm, out_hbm,
             cnt0_hbm, cnt1_hbm,
             sidx, didx, bufa, bufb, acc, sema, semb, onesb, accc,
             cbuf) = refs
        else:
            (x_hbm, srcw, dstw, zf_hbm, out_hbm,
             sidx, didx, bufa, bufb, acc, sema, semb) = refs
        cid = lax.axis_index("c")
        sid = lax.axis_index("s")
        wid = sid * NC + cid

        base = sid * RPTA
        tail = NS * RPTA
        pltpu.sync_copy(zf_hbm.at[pl.ds(base, RPTA)],
                        acc.at[pl.ds(base, RPTA)])
        if with_count:
            pltpu.sync_copy(zc_hbm.at[pl.ds(base, RPTA)], cbuf)
            pltpu.sync_copy(cbuf, accc.at[pl.ds(base, RPTA)])
            pltpu.sync_copy(ones_hbm, onesb)

        @pl.when(sid == NS - 1)
        def _zero_tail():
            pltpu.sync_copy(zf_hbm.at[pl.ds(tail, TAIL)],
                            acc.at[pl.ds(tail, TAIL)])
            if with_count:
                pltpu.sync_copy(cbuf.at[pl.ds(0, TAIL)],
                                accc.at[pl.ds(tail, TAIL)])

        plsc.subcore_barrier()

        def wait_a():
            pltpu.make_async_copy(x_hbm.at[sidx.at[0]], bufa, sema).wait()

        def wait_b():
            pltpu.make_async_copy(x_hbm.at[sidx.at[0]], bufb, semb).wait()

        def scat(buf, j):
            pltpu.sync_copy(buf, acc.at[didx.at[j]], add=True)
            if with_count:
                pltpu.sync_copy(onesb, accc.at[didx.at[j]], add=True)

        def super_body(sb, carry):
            pltpu.sync_copy(srcw.at[wid, sb], sidx)
            pltpu.sync_copy(dstw.at[wid, sb], didx)
            pltpu.async_copy(x_hbm.at[sidx.at[0]], bufa, sema)

            def pair(jj, c):
                a = jj * 2
                pltpu.async_copy(x_hbm.at[sidx.at[a + 1]], bufb, semb)
                wait_a()
                scat(bufa, a)
                pltpu.async_copy(x_hbm.at[sidx.at[a + 2]], bufa, sema)
                wait_b()
                scat(bufb, a + 1)
                return c

            if SB % 2:
                lax.fori_loop(0, (SB - 1) // 2, pair, 0)
                wait_a()
                scat(bufa, SB - 1)
            else:
                lax.fori_loop(0, (SB - 2) // 2, pair, 0)
                pltpu.async_copy(x_hbm.at[sidx.at[SB - 1]], bufb, semb)
                wait_a()
                scat(bufa, SB - 2)
                wait_b()
                scat(bufb, SB - 1)
            return carry

        lax.fori_loop(0, NSB, super_body, 0)

        plsc.subcore_barrier()
        pltpu.sync_copy(acc.at[pl.ds(base, RPTA)],
                        out_hbm.at[cid, pl.ds(base, RPTA)])
        if with_count:
            pltpu.sync_copy(accc.at[pl.ds(base, RPTA)], cbuf)

            @pl.when(cid == 0)
            def _write_cnt0():
                pltpu.sync_copy(cbuf, cnt0_hbm.at[pl.ds(base, RPTA)])

            @pl.when(cid == 1)
            def _write_cnt1():
                pltpu.sync_copy(cbuf, cnt1_hbm.at[pl.ds(base, RPTA)])

        @pl.when(sid == NS - 1)
        def _write_tail():
            pltpu.sync_copy(acc.at[pl.ds(tail, TAIL)],
                            out_hbm.at[cid, pl.ds(tail, TAIL)])
            if with_count:
                pltpu.sync_copy(accc.at[pl.ds(tail, TAIL)],
                                cbuf.at[pl.ds(0, TAIL)])

                @pl.when(cid == 0)
                def _write_cnt0_tail():
                    pltpu.sync_copy(cbuf.at[pl.ds(0, TAIL)],
                                    cnt0_hbm.at[pl.ds(tail, TAIL)])

                @pl.when(cid == 1)
                def _write_cnt1_tail():
                    pltpu.sync_copy(cbuf.at[pl.ds(0, TAIL)],
                                    cnt1_hbm.at[pl.ds(tail, TAIL)])

    return pl.kernel(body, out_type=out_type, mesh=mesh,
                     scratch_types=scratch)


_sc_agg_count = _make_sc_agg(True)
_sc_agg_plain = _make_sc_agg(False)


_HALF_BLKS = N_SRC_ROWS // ROWB


def _proj_body(xs_ref, xt_ref, ws_ref, bs_ref, wt_ref, bt_ref, o_ref):
    use_src = pl.program_id(0) < _HALF_BLKS
    x = jnp.where(use_src, xs_ref[...], xt_ref[...])
    w = jnp.where(use_src, ws_ref[...], wt_ref[...])
    b = jnp.where(use_src, bs_ref[...], bt_ref[...])
    o_ref[...] = jnp.dot(x, w, preferred_element_type=jnp.float32) + b


_proj = pl.pallas_call(
    _proj_body,
    grid=(GRID,),
    in_specs=[
        pl.BlockSpec((ROWB, HD), lambda i: (jnp.minimum(i, _HALF_BLKS - 1), 0)),
        pl.BlockSpec((ROWB, HD),
                     lambda i: (jnp.maximum(i - _HALF_BLKS, 0), 0)),
        pl.BlockSpec((HD, HD), lambda i: (0, 0)),
        pl.BlockSpec((1, HD), lambda i: (0, 0)),
        pl.BlockSpec((HD, HD), lambda i: (0, 0)),
        pl.BlockSpec((1, HD), lambda i: (0, 0)),
    ],
    out_specs=pl.BlockSpec((ROWB, HD), lambda i: (i, 0)),
    out_shape=jax.ShapeDtypeStruct((NNODE, HD), jnp.float32),
)


def _layer_body(relu, p_ref, c0, c1, h_ref, wl, bl, wr, o_ref):
    cnt = jnp.maximum(c0[...] + c1[...], 1.0)
    agg = (p_ref[0] + p_ref[1]) / cnt
    out = (jnp.dot(agg, wl[...], preferred_element_type=jnp.float32)
           + bl[...]
           + jnp.dot(h_ref[...], wr[...], preferred_element_type=jnp.float32))
    o_ref[...] = jnp.maximum(out, 0.0) if relu else out


def _make_layer(relu):
    return pl.pallas_call(
        functools.partial(_layer_body, relu),
        grid=(GRID,),
        in_specs=[
            pl.BlockSpec((2, ROWB, HD), lambda i: (0, i, 0)),
            pl.BlockSpec((ROWB, 1), lambda i: (i, 0)),
            pl.BlockSpec((ROWB, 1), lambda i: (i, 0)),
            pl.BlockSpec((ROWB, HD), lambda i: (i, 0)),
            pl.BlockSpec((HD, HD), lambda i: (0, 0)),
            pl.BlockSpec((1, HD), lambda i: (0, 0)),
            pl.BlockSpec((HD, HD), lambda i: (0, 0)),
        ],
        out_specs=pl.BlockSpec((ROWB, HD), lambda i: (i, 0)),
        out_shape=jax.ShapeDtypeStruct((NNODE, HD), jnp.float32),
    )


_layer0 = _make_layer(True)
_layer1 = _make_layer(False)


def kernel(src_x, tgt_x, edge_index, W_src, b_src, W_tgt, b_tgt,
           Wl0, bl0, Wr0, Wl1, bl1, Wr1):
    apad = jnp.arange(NW * PADW, dtype=jnp.int32)
    pad_src = ((apad * 13) % NNODE).reshape(NW, PADW)
    pad_dst = (NNODE + (apad % NDUM)).reshape(NW, PADW)
    srcw = jnp.concatenate(
        [edge_index[0].reshape(NW, EPW), pad_src], axis=1
    ).reshape(NW, NSB, SB, CH)
    dstw = jnp.concatenate(
        [edge_index[1].reshape(NW, EPW), pad_dst], axis=1
    ).reshape(NW, NSB, SB, CH)
    zf = jnp.zeros((NNODE, HD), jnp.float32)
    zc = jnp.zeros((NNODE,), jnp.float32)
    ones = jnp.ones((CH,), jnp.float32)

    x = _proj(src_x, tgt_x, W_src.T, b_src[None], W_tgt.T, b_tgt[None])
    p, c0, c1 = _sc_agg_count(x, srcw, dstw, zf, zc, ones)
    c0r = c0.reshape(NNODE, 1)
    c1r = c1.reshape(NNODE, 1)
    h1 = _layer0(p, c0r, c1r, x, Wl0.T, bl0[None], Wr0.T)
    (p2,) = _sc_agg_plain(h1, srcw, dstw, zf)
    h2 = _layer1(p2, c0r, c1r, h1, Wl1.T, bl1[None], Wr1.T)
    return h2


# speedup vs baseline: 11.1840x; 1.0818x over previous
import functools

import jax
import jax.numpy as jnp
from jax import lax
from jax.experimental import pallas as pl
from jax.experimental.pallas import tpu as pltpu
from jax.experimental.pallas import tpu_sc as plsc

N_SRC_ROWS = 5000
NNODE = 10000
NEDGE = 320000
HD = 128
NC = 2
NS = 16
NW = NC * NS
EPW = NEDGE // NW
CH = 96
PADW = 80
EPWP = EPW + PADW
NCHK = EPWP // CH
NSB = 5
SB = NCHK // NSB
NDUM = 256
NND = NNODE + NDUM
RPTA = 624
TAIL = NNODE - NS * RPTA
CW = 16
ROWB = 2000
GRID = NNODE // ROWB
PROJB = 1000
PGRID = NNODE // PROJB


def _make_sc_agg(with_count: bool):
    mesh = plsc.VectorSubcoreMesh(
        core_axis_name="c", subcore_axis_name="s",
        num_cores=NC, num_subcores=NS)
    out_type = [jax.ShapeDtypeStruct((NC, NNODE, HD), jnp.float32)]
    scratch = [
        pltpu.VMEM((SB, CH), jnp.int32),
        pltpu.VMEM((SB, CH), jnp.int32),
        pltpu.VMEM((CH, HD), jnp.float32),
        pltpu.VMEM((CH, HD), jnp.float32),
        pltpu.VMEM((CH, HD), jnp.float32),
        pltpu.VMEM_SHARED((NND, HD), jnp.float32),
        pltpu.SemaphoreType.DMA,
        pltpu.SemaphoreType.DMA,
        pltpu.SemaphoreType.DMA,
    ]
    if with_count:
        out_type.append(jax.ShapeDtypeStruct((NNODE,), jnp.float32))
        out_type.append(jax.ShapeDtypeStruct((NNODE,), jnp.float32))
        scratch += [
            pltpu.VMEM((CH,), jnp.float32),
            pltpu.VMEM_SHARED((NND,), jnp.float32),
            pltpu.VMEM((RPTA,), jnp.float32),
        ]

    def body(*refs):
        if with_count:
            (x_hbm, srcw, dstw, zf_hbm, zc_hbm, ones_hbm, out_hbm,
             cnt0_hbm, cnt1_hbm,
             sidx, didx, bufa, bufb, bufc, acc, sema, semb, semc,
             onesb, accc, cbuf) = refs
        else:
            (x_hbm, srcw, dstw, zf_hbm, out_hbm,
             sidx, didx, bufa, bufb, bufc, acc, sema, semb, semc) = refs
        cid = lax.axis_index("c")
        sid = lax.axis_index("s")
        wid = sid * NC + cid

        base = sid * RPTA
        tail = NS * RPTA
        pltpu.sync_copy(zf_hbm.at[pl.ds(base, RPTA)],
                        acc.at[pl.ds(base, RPTA)])
        if with_count:
            pltpu.sync_copy(zc_hbm.at[pl.ds(base, RPTA)], cbuf)
            pltpu.sync_copy(cbuf, accc.at[pl.ds(base, RPTA)])
            pltpu.sync_copy(ones_hbm, onesb)

        @pl.when(sid == NS - 1)
        def _zero_tail():
            pltpu.sync_copy(zf_hbm.at[pl.ds(tail, TAIL)],
                            acc.at[pl.ds(tail, TAIL)])
            if with_count:
                pltpu.sync_copy(cbuf.at[pl.ds(0, TAIL)],
                                accc.at[pl.ds(tail, TAIL)])

        plsc.subcore_barrier()

        def fire(buf, sem, j):
            pltpu.async_copy(x_hbm.at[sidx.at[j]], buf, sem)

        def wait(buf, sem):
            pltpu.make_async_copy(x_hbm.at[sidx.at[0]], buf, sem).wait()

        def scat(buf, j):
            pltpu.sync_copy(buf, acc.at[didx.at[j]], add=True)
            if with_count:
                pltpu.sync_copy(onesb, accc.at[didx.at[j]], add=True)

        assert SB % 3 == 0
        def super_body(sb, carry):
            pltpu.sync_copy(srcw.at[wid, sb], sidx)
            pltpu.sync_copy(dstw.at[wid, sb], didx)
            fire(bufa, sema, 0)
            fire(bufb, semb, 1)

            def trip(jj, c):
                t = jj * 3
                fire(bufc, semc, t + 2)
                wait(bufa, sema)
                scat(bufa, t)
                fire(bufa, sema, t + 3)
                wait(bufb, semb)
                scat(bufb, t + 1)
                fire(bufb, semb, t + 4)
                wait(bufc, semc)
                scat(bufc, t + 2)
                return c

            lax.fori_loop(0, (SB - 3) // 3, trip, 0)
            fire(bufc, semc, SB - 1)
            wait(bufa, sema)
            scat(bufa, SB - 3)
            wait(bufb, semb)
            scat(bufb, SB - 2)
            wait(bufc, semc)
            scat(bufc, SB - 1)
            return carry

        lax.fori_loop(0, NSB, super_body, 0)

        plsc.subcore_barrier()
        pltpu.sync_copy(acc.at[pl.ds(base, RPTA)],
                        out_hbm.at[cid, pl.ds(base, RPTA)])
        if with_count:
            pltpu.sync_copy(accc.at[pl.ds(base, RPTA)], cbuf)

            @pl.when(cid == 0)
            def _write_cnt0():
                pltpu.sync_copy(cbuf, cnt0_hbm.at[pl.ds(base, RPTA)])

            @pl.when(cid == 1)
            def _write_cnt1():
                pltpu.sync_copy(cbuf, cnt1_hbm.at[pl.ds(base, RPTA)])

        @pl.when(sid == NS - 1)
        def _write_tail():
            pltpu.sync_copy(acc.at[pl.ds(tail, TAIL)],
                            out_hbm.at[cid, pl.ds(tail, TAIL)])
            if with_count:
                pltpu.sync_copy(accc.at[pl.ds(tail, TAIL)],
                                cbuf.at[pl.ds(0, TAIL)])

                @pl.when(cid == 0)
                def _write_cnt0_tail():
                    pltpu.sync_copy(cbuf.at[pl.ds(0, TAIL)],
                                    cnt0_hbm.at[pl.ds(tail, TAIL)])

                @pl.when(cid == 1)
                def _write_cnt1_tail():
                    pltpu.sync_copy(cbuf.at[pl.ds(0, TAIL)],
                                    cnt1_hbm.at[pl.ds(tail, TAIL)])

    return pl.kernel(body, out_type=out_type, mesh=mesh,
                     scratch_types=scratch)


_sc_agg_count = _make_sc_agg(True)
_sc_agg_plain = _make_sc_agg(False)


_HALF_BLKS = N_SRC_ROWS // PROJB


def _proj_body(xs_ref, xt_ref, ws_ref, bs_ref, wt_ref, bt_ref, o_ref):
    use_src = pl.program_id(0) < _HALF_BLKS
    x = jnp.where(use_src, xs_ref[...], xt_ref[...])
    w = jnp.where(use_src, ws_ref[...], wt_ref[...])
    b = jnp.where(use_src, bs_ref[...], bt_ref[...])
    o_ref[...] = jnp.dot(x, w, preferred_element_type=jnp.float32) + b


_proj = pl.pallas_call(
    _proj_body,
    grid=(PGRID,),
    in_specs=[
        pl.BlockSpec((PROJB, HD),
                     lambda i: (jnp.minimum(i, _HALF_BLKS - 1), 0)),
        pl.BlockSpec((PROJB, HD),
                     lambda i: (jnp.maximum(i - _HALF_BLKS, 0), 0)),
        pl.BlockSpec((HD, HD), lambda i: (0, 0)),
        pl.BlockSpec((1, HD), lambda i: (0, 0)),
        pl.BlockSpec((HD, HD), lambda i: (0, 0)),
        pl.BlockSpec((1, HD), lambda i: (0, 0)),
    ],
    out_specs=pl.BlockSpec((PROJB, HD), lambda i: (i, 0)),
    out_shape=jax.ShapeDtypeStruct((NNODE, HD), jnp.float32),
)


def _layer_body(relu, p_ref, c0, c1, h_ref, wl, bl, wr, o_ref):
    cnt = jnp.maximum(c0[...] + c1[...], 1.0)
    agg = (p_ref[0] + p_ref[1]) / cnt
    out = (jnp.dot(agg, wl[...], preferred_element_type=jnp.float32)
           + bl[...]
           + jnp.dot(h_ref[...], wr[...], preferred_element_type=jnp.float32))
    o_ref[...] = jnp.maximum(out, 0.0) if relu else out


def _make_layer(relu):
    return pl.pallas_call(
        functools.partial(_layer_body, relu),
        grid=(GRID,),
        in_specs=[
            pl.BlockSpec((2, ROWB, HD), lambda i: (0, i, 0)),
            pl.BlockSpec((ROWB, 1), lambda i: (i, 0)),
            pl.BlockSpec((ROWB, 1), lambda i: (i, 0)),
            pl.BlockSpec((ROWB, HD), lambda i: (i, 0)),
            pl.BlockSpec((HD, HD), lambda i: (0, 0)),
            pl.BlockSpec((1, HD), lambda i: (0, 0)),
            pl.BlockSpec((HD, HD), lambda i: (0, 0)),
        ],
        out_specs=pl.BlockSpec((ROWB, HD), lambda i: (i, 0)),
        out_shape=jax.ShapeDtypeStruct((NNODE, HD), jnp.float32),
    )


_layer0 = _make_layer(True)
_layer1 = _make_layer(False)


def kernel(src_x, tgt_x, edge_index, W_src, b_src, W_tgt, b_tgt,
           Wl0, bl0, Wr0, Wl1, bl1, Wr1):
    apad = jnp.arange(NW * PADW, dtype=jnp.int32)
    pad_src = ((apad * 13) % NNODE).reshape(NW, PADW)
    pad_dst = (NNODE + (apad % NDUM)).reshape(NW, PADW)
    srcw = jnp.concatenate(
        [edge_index[0].reshape(NW, EPW), pad_src], axis=1
    ).reshape(NW, NSB, SB, CH)
    dstw = jnp.concatenate(
        [edge_index[1].reshape(NW, EPW), pad_dst], axis=1
    ).reshape(NW, NSB, SB, CH)
    zf = jnp.zeros((NNODE, HD), jnp.float32)
    zc = jnp.zeros((NNODE,), jnp.float32)
    ones = jnp.ones((CH,), jnp.float32)

    x = _proj(src_x, tgt_x, W_src.T, b_src[None], W_tgt.T, b_tgt[None])
    p, c0, c1 = _sc_agg_count(x, srcw, dstw, zf, zc, ones)
    c0r = c0.reshape(NNODE, 1)
    c1r = c1.reshape(NNODE, 1)
    h1 = _layer0(p, c0r, c1r, x, Wl0.T, bl0[None], Wr0.T)
    (p2,) = _sc_agg_plain(h1, srcw, dstw, zf)
    h2 = _layer1(p2, c0r, c1r, h1, Wl1.T, bl1[None], Wr1.T)
    return h2
